# baseline (speedup 1.0000x reference)
import functools

import jax
import jax.numpy as jnp
from jax import lax
from jax.experimental import pallas as pl
from jax.experimental.pallas import tpu as pltpu
from jax.experimental.pallas import tpu_sc as plsc

_DN0 = (((0,), (0,)), ((), ()))
_DN01 = (((0,), (1,)), ((), ()))

_EDGE_BLOCK = 16000
_SC_WINDOW = 3200


def _prep_nodes_body(x_ref, w_in_ref, b_in_ref, w_top_ref, hv0t_ref, g0t_ref):
    hv0t = lax.dot_general(
        w_in_ref[...], x_ref[...], _DN01, preferred_element_type=jnp.float32
    ) + b_in_ref[...]
    hv0t_ref[...] = hv0t
    g0t_ref[...] = lax.dot_general(
        w_top_ref[...], hv0t, _DN0, preferred_element_type=jnp.float32
    )


def _prep_edges_body(ea_ref, w2_ref, b2_ref, eat0_ref, eat1_ref):
    m = lax.dot_general(
        w2_ref[...], ea_ref[...], _DN01, preferred_element_type=jnp.float32
    ) + b2_ref[...]
    half = m.shape[0] // 2
    eat0_ref[...] = m[:half]
    eat1_ref[...] = m[half:]


def _mid_body(pt_ref, hvt_ref, w_top_ref, hv1t_ref, g1t_ref):
    pooled = jnp.max(pt_ref[...], axis=0)
    hv1t = jnp.maximum(pooled, 0.0) + hvt_ref[...]
    hv1t_ref[...] = hv1t
    g1t_ref[...] = lax.dot_general(
        w_top_ref[...], hv1t, _DN0, preferred_element_type=jnp.float32
    )


def _fin_body(pt_ref, hvt_ref, w_r_ref, b_r_ref, out_ref):
    pooled = jnp.max(pt_ref[...], axis=0)
    hv2t = jnp.maximum(pooled, 0.0) + hvt_ref[...]
    out_ref[...] = jnp.maximum(
        lax.dot_general(hv2t, w_r_ref[...], _DN0, preferred_element_type=jnp.float32)
        + b_r_ref[...],
        0.0,
    )


_K = 4
_C = 4

_GDN = lax.GatherDimensionNumbers(
    offset_dims=(), collapsed_slice_dims=(0,), start_index_map=(0,)
)


def _take(x, idx):
    return lax.gather(
        x,
        idx[:, None],
        dimension_numbers=_GDN,
        slice_sizes=(1,),
        mode=lax.GatherScatterMode.PROMISE_IN_BOUNDS,
    )


def _sc_round(gt, eat, src, dst):
    n = gt.shape[1]
    e = src.shape[0]
    w = _SC_WINDOW
    ec = e // _C
    mesh = plsc.VectorSubcoreMesh(core_axis_name="c", subcore_axis_name="s")

    @functools.partial(
        pl.kernel,
        out_type=jax.ShapeDtypeStruct((_C * 32, n), jnp.float32),
        mesh=mesh,
        compiler_params=pltpu.CompilerParams(needs_layout_passes=False),
        scratch_types=[pltpu.VMEM((n,), jnp.float32)] * 8
        + [pltpu.VMEM((w,), jnp.int32)] * 4
        + [pltpu.VMEM((w,), jnp.float32)] * 8,
    )
    def k(gt_h, eat_h, src_h, dst_h, out_h, *scr):
        gs = scr[0:4]
        accs = scr[4:8]
        src_v, dst_v, sdst_v, mend_v = scr[8:12]
        eas = scr[12:16]
        zs = scr[16:20]
        c = lax.axis_index("c")
        s = lax.axis_index("s")
        wid = s * 2 + c
        grp = wid // _C
        j = wid % _C
        f0 = grp * _K

        for ki in range(_K):
            pltpu.sync_copy(gt_h.at[f0 + ki], gs[ki])

        neg_inf = jnp.full((16,), -jnp.inf, dtype=jnp.float32)

        @plsc.parallel_loop(0, n // 16, unroll=4)
        def _init(i):
            for ki in range(_K):
                accs[ki][pl.ds(i * 16, 16)] = neg_inf

        lane = lax.iota(jnp.int32, 16)
        _SHIFT_UP = jnp.minimum(lane + 1, 15)
        _SHIFT_DN = {sh: jnp.maximum(lane - sh, 0) for sh in (1, 2, 4, 8)}
        base0 = j * ec

        def win_body(wi, carry):
            base = pl.multiple_of(base0 + wi * w, 128)
            pltpu.sync_copy(src_h.at[pl.ds(base, w)], src_v)
            pltpu.sync_copy(dst_h.at[pl.ds(base, w)], dst_v)
            for ki in range(_K):
                pltpu.sync_copy(eat_h.at[f0 + ki, pl.ds(base, w)], eas[ki])

            def _zloop(i, cz):
                off = i * 16
                sv = src_v[pl.ds(off, 16)]
                dv = dst_v[pl.ds(off, 16)]
                sd, p = plsc.sort_key_val(dv, lane)
                nxt = _take(sd, _SHIFT_UP)
                is_end = (sd != nxt) | (lane == 15)
                sdst_v[pl.ds(off, 16)] = sd
                mend_v[pl.ds(off, 16)] = jnp.where(is_end, 1, 0)
                eqs = []
                for sh in (1, 2, 4, 8):
                    prev = _take(sd, _SHIFT_DN[sh])
                    eqs.append((sd == prev) & (lane >= sh))
                for ki in range(_K):
                    z = plsc.load_gather(gs[ki], [sv]) + eas[ki][pl.ds(off, 16)]
                    zp = _take(z, p)
                    for sh, eq in zip((1, 2, 4, 8), eqs):
                        zsh = _take(zp, _SHIFT_DN[sh])
                        zp = jnp.maximum(zp, jnp.where(eq, zsh, -jnp.inf))
                    zs[ki][pl.ds(off, 16)] = zp
                return cz

            lax.fori_loop(0, w // 16, _zloop, 0)

            def vec_body(i, carry2):
                off = i * 16
                sd = sdst_v[pl.ds(off, 16)]
                mk = mend_v[pl.ds(off, 16)] > 0
                for ki in range(_K):
                    zf = zs[ki][pl.ds(off, 16)]
                    cur = plsc.load_gather(accs[ki], [sd], mask=mk)
                    plsc.store_scatter(
                        accs[ki], [sd], jnp.maximum(cur, zf), mask=mk
                    )
                return carry2

            lax.fori_loop(0, w // 16, vec_body, 0)
            return carry

        lax.fori_loop(0, ec // w, win_body, 0)
        for ki in range(_K):
            pltpu.sync_copy(accs[ki], out_h.at[j * 32 + f0 + ki])

    return k(gt, eat, src, dst).reshape(_C, 32, n)


def kernel(x, edge_index, edge_attr, W_in, b_in, W_u0, b_u0, W_u1, b_u1, W_r, b_r):
    n, _ = x.shape
    e = edge_index.shape[1]
    out_dim = W_in.shape[1]

    src = edge_index[0].astype(jnp.int32)
    dst = edge_index[1].astype(jnp.int32)

    w2 = jnp.concatenate([W_u0[out_dim:], W_u1[out_dim:]], axis=1)
    b2 = jnp.concatenate([b_u0, b_u1]).reshape(-1, 1)

    hv0t, g0t = pl.pallas_call(
        _prep_nodes_body,
        out_shape=[jax.ShapeDtypeStruct((out_dim, n), jnp.float32)] * 2,
    )(x, W_in, b_in.reshape(-1, 1), W_u0[:out_dim])

    de = edge_attr.shape[1]
    blk = _EDGE_BLOCK
    eat0, eat1 = pl.pallas_call(
        _prep_edges_body,
        grid=(e // blk,),
        in_specs=[
            pl.BlockSpec((blk, de), lambda i: (i, 0)),
            pl.BlockSpec((de, 2 * out_dim), lambda i: (0, 0)),
            pl.BlockSpec((2 * out_dim, 1), lambda i: (0, 0)),
        ],
        out_specs=[
            pl.BlockSpec((out_dim, blk), lambda i: (0, i)),
            pl.BlockSpec((out_dim, blk), lambda i: (0, i)),
        ],
        out_shape=[jax.ShapeDtypeStruct((out_dim, e), jnp.float32)] * 2,
    )(edge_attr, w2, b2)

    p0 = _sc_round(g0t, eat0, src, dst)

    hv1t, g1t = pl.pallas_call(
        _mid_body,
        out_shape=[jax.ShapeDtypeStruct((out_dim, n), jnp.float32)] * 2,
    )(p0, hv0t, W_u1[:out_dim])

    p1 = _sc_round(g1t, eat1, src, dst)

    out = pl.pallas_call(
        _fin_body,
        out_shape=jax.ShapeDtypeStruct((n, out_dim), jnp.float32),
    )(p1, hv1t, W_r, b_r.reshape(1, -1))

    return out


# speedup vs baseline: 1.3667x; 1.3667x over previous
import functools

import jax
import jax.numpy as jnp
from jax import lax
from jax.experimental import pallas as pl
from jax.experimental.pallas import tpu as pltpu
from jax.experimental.pallas import tpu_sc as plsc

_DN0 = (((0,), (0,)), ((), ()))
_DN01 = (((0,), (1,)), ((), ()))

_EDGE_BLOCK = 16000
_SC_WINDOW = 3200


def _prep_nodes_body(x_ref, w_in_ref, b_in_ref, w_top_ref, hv0t_ref, g0t_ref):
    hv0t = lax.dot_general(
        w_in_ref[...], x_ref[...], _DN01, preferred_element_type=jnp.float32
    ) + b_in_ref[...]
    hv0t_ref[...] = hv0t
    g0t_ref[...] = lax.dot_general(
        w_top_ref[...], hv0t, _DN0, preferred_element_type=jnp.float32
    )


def _prep_edges_body(ea_ref, w2_ref, b2_ref, eat0_ref, eat1_ref):
    m = lax.dot_general(
        w2_ref[...], ea_ref[...], _DN01, preferred_element_type=jnp.float32
    ) + b2_ref[...]
    half = m.shape[0] // 2
    eat0_ref[...] = m[:half]
    eat1_ref[...] = m[half:]


def _mid_body(pt_ref, hvt_ref, w_top_ref, hv1t_ref, g1t_ref):
    pooled = jnp.max(pt_ref[...], axis=0)
    hv1t = jnp.maximum(pooled, 0.0) + hvt_ref[...]
    hv1t_ref[...] = hv1t
    g1t_ref[...] = lax.dot_general(
        w_top_ref[...], hv1t, _DN0, preferred_element_type=jnp.float32
    )


def _fin_body(pt_ref, hvt_ref, w_r_ref, b_r_ref, out_ref):
    pooled = jnp.max(pt_ref[...], axis=0)
    hv2t = jnp.maximum(pooled, 0.0) + hvt_ref[...]
    out_ref[...] = jnp.maximum(
        lax.dot_general(hv2t, w_r_ref[...], _DN0, preferred_element_type=jnp.float32)
        + b_r_ref[...],
        0.0,
    )


_K = 4
_C = 4

_GDN = lax.GatherDimensionNumbers(
    offset_dims=(), collapsed_slice_dims=(0,), start_index_map=(0,)
)


def _take(x, idx):
    return lax.gather(
        x,
        idx[:, None],
        dimension_numbers=_GDN,
        slice_sizes=(1,),
        mode=lax.GatherScatterMode.PROMISE_IN_BOUNDS,
    )


def _sc_round(gt, eat, src, dst):
    n = gt.shape[1]
    e = src.shape[0]
    w = _SC_WINDOW
    ec = e // _C
    mesh = plsc.VectorSubcoreMesh(core_axis_name="c", subcore_axis_name="s")

    @functools.partial(
        pl.kernel,
        out_type=jax.ShapeDtypeStruct((_C * 32, n), jnp.float32),
        mesh=mesh,
        compiler_params=pltpu.CompilerParams(needs_layout_passes=False),
        scratch_types=[pltpu.VMEM((n,), jnp.float32)] * 8
        + [pltpu.VMEM((w,), jnp.int32)] * 4
        + [pltpu.VMEM((w,), jnp.float32)] * 8,
    )
    def k(gt_h, eat_h, src_h, dst_h, out_h, *scr):
        gs = scr[0:4]
        accs = scr[4:8]
        src_v, dst_v, sdst_v, mend_v = scr[8:12]
        eas = scr[12:16]
        zs = scr[16:20]
        c = lax.axis_index("c")
        s = lax.axis_index("s")
        wid = s * 2 + c
        grp = wid // _C
        j = wid % _C
        f0 = grp * _K

        for ki in range(_K):
            pltpu.sync_copy(gt_h.at[f0 + ki], gs[ki])

        neg_inf = jnp.full((16,), -jnp.inf, dtype=jnp.float32)

        @plsc.parallel_loop(0, n // 16, unroll=4)
        def _init(i):
            for ki in range(_K):
                accs[ki][pl.ds(i * 16, 16)] = neg_inf

        lane = lax.iota(jnp.int32, 16)
        _SHIFT_UP = jnp.minimum(lane + 1, 15)
        _SHIFT_DN = {sh: jnp.maximum(lane - sh, 0) for sh in (1, 2, 4, 8)}
        base0 = j * ec

        def win_body(wi, carry):
            base = pl.multiple_of(base0 + wi * w, 128)
            pltpu.sync_copy(src_h.at[pl.ds(base, w)], src_v)
            pltpu.sync_copy(dst_h.at[pl.ds(base, w)], dst_v)
            for ki in range(_K):
                pltpu.sync_copy(eat_h.at[f0 + ki, pl.ds(base, w)], eas[ki])

            @plsc.parallel_loop(0, w // 16, unroll=2)
            def _zloop(i):
                off = i * 16
                sv = src_v[pl.ds(off, 16)]
                dv = dst_v[pl.ds(off, 16)]
                sd, p = plsc.sort_key_val(dv, lane)
                nxt = _take(sd, _SHIFT_UP)
                is_end = (sd != nxt) | (lane == 15)
                sdst_v[pl.ds(off, 16)] = sd
                mend_v[pl.ds(off, 16)] = jnp.where(is_end, 1, 0)
                eqs = []
                for sh in (1, 2, 4, 8):
                    prev = _take(sd, _SHIFT_DN[sh])
                    eqs.append((sd == prev) & (lane >= sh))
                for ki in range(_K):
                    z = plsc.load_gather(gs[ki], [sv]) + eas[ki][pl.ds(off, 16)]
                    zp = _take(z, p)
                    for sh, eq in zip((1, 2, 4, 8), eqs):
                        zsh = _take(zp, _SHIFT_DN[sh])
                        zp = jnp.maximum(zp, jnp.where(eq, zsh, -jnp.inf))
                    zs[ki][pl.ds(off, 16)] = zp

            def vec_body(i, carry2):
                off = i * 16
                sd = sdst_v[pl.ds(off, 16)]
                mk = mend_v[pl.ds(off, 16)] > 0
                for ki in range(_K):
                    zf = zs[ki][pl.ds(off, 16)]
                    cur = plsc.load_gather(accs[ki], [sd], mask=mk)
                    plsc.store_scatter(
                        accs[ki], [sd], jnp.maximum(cur, zf), mask=mk
                    )
                return carry2

            lax.fori_loop(0, w // 16, vec_body, 0)
            return carry

        lax.fori_loop(0, ec // w, win_body, 0)
        for ki in range(_K):
            pltpu.sync_copy(accs[ki], out_h.at[j * 32 + f0 + ki])

    return k(gt, eat, src, dst).reshape(_C, 32, n)


def kernel(x, edge_index, edge_attr, W_in, b_in, W_u0, b_u0, W_u1, b_u1, W_r, b_r):
    n, _ = x.shape
    e = edge_index.shape[1]
    out_dim = W_in.shape[1]

    src = edge_index[0].astype(jnp.int32)
    dst = edge_index[1].astype(jnp.int32)

    w2 = jnp.concatenate([W_u0[out_dim:], W_u1[out_dim:]], axis=1)
    b2 = jnp.concatenate([b_u0, b_u1]).reshape(-1, 1)

    hv0t, g0t = pl.pallas_call(
        _prep_nodes_body,
        out_shape=[jax.ShapeDtypeStruct((out_dim, n), jnp.float32)] * 2,
    )(x, W_in, b_in.reshape(-1, 1), W_u0[:out_dim])

    de = edge_attr.shape[1]
    blk = _EDGE_BLOCK
    eat0, eat1 = pl.pallas_call(
        _prep_edges_body,
        grid=(e // blk,),
        in_specs=[
            pl.BlockSpec((blk, de), lambda i: (i, 0)),
            pl.BlockSpec((de, 2 * out_dim), lambda i: (0, 0)),
            pl.BlockSpec((2 * out_dim, 1), lambda i: (0, 0)),
        ],
        out_specs=[
            pl.BlockSpec((out_dim, blk), lambda i: (0, i)),
            pl.BlockSpec((out_dim, blk), lambda i: (0, i)),
        ],
        out_shape=[jax.ShapeDtypeStruct((out_dim, e), jnp.float32)] * 2,
    )(edge_attr, w2, b2)

    p0 = _sc_round(g0t, eat0, src, dst)

    hv1t, g1t = pl.pallas_call(
        _mid_body,
        out_shape=[jax.ShapeDtypeStruct((out_dim, n), jnp.float32)] * 2,
    )(p0, hv0t, W_u1[:out_dim])

    p1 = _sc_round(g1t, eat1, src, dst)

    out = pl.pallas_call(
        _fin_body,
        out_shape=jax.ShapeDtypeStruct((n, out_dim), jnp.float32),
    )(p1, hv1t, W_r, b_r.reshape(1, -1))

    return out


# speedup vs baseline: 1.6895x; 1.2362x over previous
import functools

import jax
import jax.numpy as jnp
from jax import lax
from jax.experimental import pallas as pl
from jax.experimental.pallas import tpu as pltpu
from jax.experimental.pallas import tpu_sc as plsc

_DN0 = (((0,), (0,)), ((), ()))
_DN01 = (((0,), (1,)), ((), ()))

_EDGE_BLOCK = 16000
_SC_WINDOW = 3200


def _prep_nodes_body(x_ref, w_in_ref, b_in_ref, w_top_ref, hv0t_ref, g0t_ref):
    hv0t = lax.dot_general(
        w_in_ref[...], x_ref[...], _DN01, preferred_element_type=jnp.float32
    ) + b_in_ref[...]
    hv0t_ref[...] = hv0t
    g0t_ref[...] = lax.dot_general(
        w_top_ref[...], hv0t, _DN0, preferred_element_type=jnp.float32
    )


def _prep_edges_body(ea_ref, w2_ref, b2_ref, eat0_ref, eat1_ref):
    m = lax.dot_general(
        w2_ref[...], ea_ref[...], _DN01, preferred_element_type=jnp.float32
    ) + b2_ref[...]
    half = m.shape[0] // 2
    eat0_ref[...] = m[:half]
    eat1_ref[...] = m[half:]


def _mid_body(pt_ref, hvt_ref, w_top_ref, hv1t_ref, g1t_ref):
    pooled = jnp.max(pt_ref[...], axis=0)[:, : hvt_ref.shape[1]]
    hv1t = jnp.maximum(pooled, 0.0) + hvt_ref[...]
    hv1t_ref[...] = hv1t
    g1t_ref[...] = lax.dot_general(
        w_top_ref[...], hv1t, _DN0, preferred_element_type=jnp.float32
    )


def _fin_body(pt_ref, hvt_ref, w_r_ref, b_r_ref, out_ref):
    pooled = jnp.max(pt_ref[...], axis=0)[:, : hvt_ref.shape[1]]
    hv2t = jnp.maximum(pooled, 0.0) + hvt_ref[...]
    out_ref[...] = jnp.maximum(
        lax.dot_general(hv2t, w_r_ref[...], _DN0, preferred_element_type=jnp.float32)
        + b_r_ref[...],
        0.0,
    )


_K = 4
_C = 4

_GDN = lax.GatherDimensionNumbers(
    offset_dims=(), collapsed_slice_dims=(0,), start_index_map=(0,)
)


def _take(x, idx):
    return lax.gather(
        x,
        idx[:, None],
        dimension_numbers=_GDN,
        slice_sizes=(1,),
        mode=lax.GatherScatterMode.PROMISE_IN_BOUNDS,
    )


def _sc_round(gt, eat, src, dst):
    n = gt.shape[1]
    e = src.shape[0]
    w = _SC_WINDOW
    ec = e // _C
    mesh = plsc.VectorSubcoreMesh(core_axis_name="c", subcore_axis_name="s")

    @functools.partial(
        pl.kernel,
        out_type=jax.ShapeDtypeStruct((_C * 32, n + 16), jnp.float32),
        mesh=mesh,
        compiler_params=pltpu.CompilerParams(needs_layout_passes=False),
        scratch_types=[pltpu.VMEM((n,), jnp.float32)] * 4
        + [pltpu.VMEM((n + 16,), jnp.float32)] * 4
        + [pltpu.VMEM((w,), jnp.int32)] * 3
        + [pltpu.VMEM((w,), jnp.float32)] * 8
        + [pltpu.SemaphoreType.DMA],
    )
    def k(gt_h, eat_h, src_h, dst_h, out_h, *scr):
        gs = scr[0:4]
        accs = scr[4:8]
        src_v, dst_v, sdst_v = scr[8:11]
        eas = scr[11:15]
        zs = scr[15:19]
        sem = scr[19]
        c = lax.axis_index("c")
        s = lax.axis_index("s")
        wid = s * 2 + c
        grp = wid // _C
        j = wid % _C
        f0 = grp * _K

        for ki in range(_K):
            pltpu.sync_copy(gt_h.at[f0 + ki], gs[ki])

        neg_inf = jnp.full((16,), -jnp.inf, dtype=jnp.float32)

        @plsc.parallel_loop(0, (n + 16) // 16, unroll=4)
        def _init(i):
            for ki in range(_K):
                accs[ki][pl.ds(i * 16, 16)] = neg_inf

        lane = lax.iota(jnp.int32, 16)
        _SHIFT_UP = jnp.minimum(lane + 1, 15)
        _SHIFT_DN = {sh: jnp.maximum(lane - sh, 0) for sh in (1, 2, 4, 8)}
        base0 = j * ec

        def win_body(wi, carry):
            base = pl.multiple_of(base0 + wi * w, 128)
            cps = [
                pltpu.async_copy(src_h.at[pl.ds(base, w)], src_v, sem),
                pltpu.async_copy(dst_h.at[pl.ds(base, w)], dst_v, sem),
            ]
            for ki in range(_K):
                cps.append(
                    pltpu.async_copy(eat_h.at[f0 + ki, pl.ds(base, w)], eas[ki], sem)
                )
            for cp in cps:
                cp.wait()

            @plsc.parallel_loop(0, w // 16, unroll=2)
            def _zloop(i):
                off = i * 16
                sv = src_v[pl.ds(off, 16)]
                dv = dst_v[pl.ds(off, 16)]
                sd, p = plsc.sort_key_val(dv, lane)
                nxt = _take(sd, _SHIFT_UP)
                is_end = (sd != nxt) | (lane == 15)
                sdst_v[pl.ds(off, 16)] = jnp.where(is_end, sd, n + lane)
                eqs = []
                for sh in (1, 2, 4, 8):
                    prev = _take(sd, _SHIFT_DN[sh])
                    eqs.append((sd == prev) & (lane >= sh))
                for ki in range(_K):
                    z = plsc.load_gather(gs[ki], [sv]) + eas[ki][pl.ds(off, 16)]
                    zp = _take(z, p)
                    for sh, eq in zip((1, 2, 4, 8), eqs):
                        zsh = _take(zp, _SHIFT_DN[sh])
                        zp = jnp.maximum(zp, jnp.where(eq, zsh, -jnp.inf))
                    zs[ki][pl.ds(off, 16)] = zp

            def vec_body(i, carry2):
                off = i * 16
                sd = sdst_v[pl.ds(off, 16)]
                for ki in range(_K):
                    zf = zs[ki][pl.ds(off, 16)]
                    cur = plsc.load_gather(accs[ki], [sd])
                    plsc.store_scatter(accs[ki], [sd], jnp.maximum(cur, zf))
                return carry2

            lax.fori_loop(0, w // 16, vec_body, 0)
            return carry

        lax.fori_loop(0, ec // w, win_body, 0)
        for ki in range(_K):
            pltpu.sync_copy(accs[ki], out_h.at[j * 32 + f0 + ki])

    return k(gt, eat, src, dst).reshape(_C, 32, n + 16)


def kernel(x, edge_index, edge_attr, W_in, b_in, W_u0, b_u0, W_u1, b_u1, W_r, b_r):
    n, _ = x.shape
    e = edge_index.shape[1]
    out_dim = W_in.shape[1]

    src = edge_index[0].astype(jnp.int32)
    dst = edge_index[1].astype(jnp.int32)

    w2 = jnp.concatenate([W_u0[out_dim:], W_u1[out_dim:]], axis=1)
    b2 = jnp.concatenate([b_u0, b_u1]).reshape(-1, 1)

    hv0t, g0t = pl.pallas_call(
        _prep_nodes_body,
        out_shape=[jax.ShapeDtypeStruct((out_dim, n), jnp.float32)] * 2,
    )(x, W_in, b_in.reshape(-1, 1), W_u0[:out_dim])

    de = edge_attr.shape[1]
    blk = _EDGE_BLOCK
    eat0, eat1 = pl.pallas_call(
        _prep_edges_body,
        grid=(e // blk,),
        in_specs=[
            pl.BlockSpec((blk, de), lambda i: (i, 0)),
            pl.BlockSpec((de, 2 * out_dim), lambda i: (0, 0)),
            pl.BlockSpec((2 * out_dim, 1), lambda i: (0, 0)),
        ],
        out_specs=[
            pl.BlockSpec((out_dim, blk), lambda i: (0, i)),
            pl.BlockSpec((out_dim, blk), lambda i: (0, i)),
        ],
        out_shape=[jax.ShapeDtypeStruct((out_dim, e), jnp.float32)] * 2,
    )(edge_attr, w2, b2)

    p0 = _sc_round(g0t, eat0, src, dst)

    hv1t, g1t = pl.pallas_call(
        _mid_body,
        out_shape=[jax.ShapeDtypeStruct((out_dim, n), jnp.float32)] * 2,
    )(p0, hv0t, W_u1[:out_dim])

    p1 = _sc_round(g1t, eat1, src, dst)

    out = pl.pallas_call(
        _fin_body,
        out_shape=jax.ShapeDtypeStruct((n, out_dim), jnp.float32),
    )(p1, hv1t, W_r, b_r.reshape(1, -1))

    return out


# speedup vs baseline: 1.7066x; 1.0101x over previous
import functools

import jax
import jax.numpy as jnp
from jax import lax
from jax.experimental import pallas as pl
from jax.experimental.pallas import tpu as pltpu
from jax.experimental.pallas import tpu_sc as plsc

_DN0 = (((0,), (0,)), ((), ()))
_DN01 = (((0,), (1,)), ((), ()))

_EDGE_BLOCK = 16000
_SC_WINDOW = 3200


def _prep_nodes_body(x_ref, w_in_ref, b_in_ref, w_top_ref, hv0t_ref, g0t_ref):
    hv0t = lax.dot_general(
        w_in_ref[...], x_ref[...], _DN01, preferred_element_type=jnp.float32
    ) + b_in_ref[...]
    hv0t_ref[...] = hv0t
    g0t_ref[...] = lax.dot_general(
        w_top_ref[...], hv0t, _DN0, preferred_element_type=jnp.float32
    )


def _prep_edges_body(ea_ref, w2_ref, b2_ref, eat0_ref, eat1_ref):
    m = lax.dot_general(
        w2_ref[...], ea_ref[...], _DN01, preferred_element_type=jnp.float32
    ) + b2_ref[...]
    half = m.shape[0] // 2
    eat0_ref[...] = m[:half]
    eat1_ref[...] = m[half:]


def _mid_body(pt_ref, hvt_ref, w_top_ref, hv1t_ref, g1t_ref):
    pooled = jnp.max(pt_ref[...], axis=0)[:, : hvt_ref.shape[1]]
    hv1t = jnp.maximum(pooled, 0.0) + hvt_ref[...]
    hv1t_ref[...] = hv1t
    g1t_ref[...] = lax.dot_general(
        w_top_ref[...], hv1t, _DN0, preferred_element_type=jnp.float32
    )


def _fin_body(pt_ref, hvt_ref, w_r_ref, b_r_ref, out_ref):
    pooled = jnp.max(pt_ref[...], axis=0)[:, : hvt_ref.shape[1]]
    hv2t = jnp.maximum(pooled, 0.0) + hvt_ref[...]
    out_ref[...] = jnp.maximum(
        lax.dot_general(hv2t, w_r_ref[...], _DN0, preferred_element_type=jnp.float32)
        + b_r_ref[...],
        0.0,
    )


_K = 4
_C = 4

_GDN = lax.GatherDimensionNumbers(
    offset_dims=(), collapsed_slice_dims=(0,), start_index_map=(0,)
)


def _take(x, idx):
    return lax.gather(
        x,
        idx[:, None],
        dimension_numbers=_GDN,
        slice_sizes=(1,),
        mode=lax.GatherScatterMode.PROMISE_IN_BOUNDS,
    )


def _sc_round(gt, eat, src, dst):
    n = gt.shape[1]
    e = src.shape[0]
    w = _SC_WINDOW
    ec = e // _C
    mesh = plsc.VectorSubcoreMesh(core_axis_name="c", subcore_axis_name="s")

    @functools.partial(
        pl.kernel,
        out_type=jax.ShapeDtypeStruct((_C * 32, n + 16), jnp.float32),
        mesh=mesh,
        compiler_params=pltpu.CompilerParams(needs_layout_passes=False),
        scratch_types=[pltpu.VMEM((n,), jnp.float32)] * 4
        + [pltpu.VMEM((n + 16,), jnp.float32)] * 4
        + [pltpu.VMEM((w,), jnp.int32)] * 3
        + [pltpu.VMEM((w,), jnp.float32)] * 8
        + [pltpu.SemaphoreType.DMA],
    )
    def k(gt_h, eat_h, src_h, dst_h, out_h, *scr):
        gs = scr[0:4]
        accs = scr[4:8]
        src_v, dst_v, sdst_v = scr[8:11]
        eas = scr[11:15]
        zs = scr[15:19]
        sem = scr[19]
        c = lax.axis_index("c")
        s = lax.axis_index("s")
        wid = s * 2 + c
        grp = wid // _C
        j = wid % _C
        f0 = grp * _K

        for ki in range(_K):
            pltpu.sync_copy(gt_h.at[f0 + ki], gs[ki])

        neg_inf = jnp.full((16,), -jnp.inf, dtype=jnp.float32)

        @plsc.parallel_loop(0, (n + 16) // 16, unroll=4)
        def _init(i):
            for ki in range(_K):
                accs[ki][pl.ds(i * 16, 16)] = neg_inf

        lane = lax.iota(jnp.int32, 16)
        _SHIFT_UP = jnp.minimum(lane + 1, 15)
        _SHIFT_DN = {sh: jnp.maximum(lane - sh, 0) for sh in (1, 2, 4, 8)}
        base0 = j * ec

        def win_body(wi, carry):
            base = pl.multiple_of(base0 + wi * w, 128)
            cps = [
                pltpu.async_copy(src_h.at[pl.ds(base, w)], src_v, sem),
                pltpu.async_copy(dst_h.at[pl.ds(base, w)], dst_v, sem),
            ]
            for ki in range(_K):
                cps.append(
                    pltpu.async_copy(eat_h.at[f0 + ki, pl.ds(base, w)], eas[ki], sem)
                )
            for cp in cps:
                cp.wait()

            @plsc.parallel_loop(0, w // 16, unroll=2)
            def _zloop(i):
                off = i * 16
                sv = src_v[pl.ds(off, 16)]
                dv = dst_v[pl.ds(off, 16)]
                sd, p = plsc.sort_key_val(dv, lane)
                nxt = _take(sd, _SHIFT_UP)
                is_end = (sd != nxt) | (lane == 15)
                sdst_v[pl.ds(off, 16)] = jnp.where(is_end, sd, n + lane)
                eqs = []
                for sh in (1, 2, 4, 8):
                    prev = _take(sd, _SHIFT_DN[sh])
                    eqs.append((sd == prev) & (lane >= sh))
                for ki in range(_K):
                    z = plsc.load_gather(gs[ki], [sv]) + eas[ki][pl.ds(off, 16)]
                    zp = _take(z, p)
                    for sh, eq in zip((1, 2, 4, 8), eqs):
                        zsh = _take(zp, _SHIFT_DN[sh])
                        zp = jnp.maximum(zp, jnp.where(eq, zsh, -jnp.inf))
                    zs[ki][pl.ds(off, 16)] = zp

            def vec_body(i, carry2):
                for u in range(2):
                    off = (i * 2 + u) * 16
                    sd = sdst_v[pl.ds(off, 16)]
                    for ki in range(_K):
                        zf = zs[ki][pl.ds(off, 16)]
                        cur = plsc.load_gather(accs[ki], [sd])
                        plsc.store_scatter(accs[ki], [sd], jnp.maximum(cur, zf))
                return carry2

            lax.fori_loop(0, w // 32, vec_body, 0)
            return carry

        lax.fori_loop(0, ec // w, win_body, 0)
        for ki in range(_K):
            pltpu.sync_copy(accs[ki], out_h.at[j * 32 + f0 + ki])

    return k(gt, eat, src, dst).reshape(_C, 32, n + 16)


def kernel(x, edge_index, edge_attr, W_in, b_in, W_u0, b_u0, W_u1, b_u1, W_r, b_r):
    n, _ = x.shape
    e = edge_index.shape[1]
    out_dim = W_in.shape[1]

    src = edge_index[0].astype(jnp.int32)
    dst = edge_index[1].astype(jnp.int32)

    w2 = jnp.concatenate([W_u0[out_dim:], W_u1[out_dim:]], axis=1)
    b2 = jnp.concatenate([b_u0, b_u1]).reshape(-1, 1)

    hv0t, g0t = pl.pallas_call(
        _prep_nodes_body,
        out_shape=[jax.ShapeDtypeStruct((out_dim, n), jnp.float32)] * 2,
    )(x, W_in, b_in.reshape(-1, 1), W_u0[:out_dim])

    de = edge_attr.shape[1]
    blk = _EDGE_BLOCK
    eat0, eat1 = pl.pallas_call(
        _prep_edges_body,
        grid=(e // blk,),
        in_specs=[
            pl.BlockSpec((blk, de), lambda i: (i, 0)),
            pl.BlockSpec((de, 2 * out_dim), lambda i: (0, 0)),
            pl.BlockSpec((2 * out_dim, 1), lambda i: (0, 0)),
        ],
        out_specs=[
            pl.BlockSpec((out_dim, blk), lambda i: (0, i)),
            pl.BlockSpec((out_dim, blk), lambda i: (0, i)),
        ],
        out_shape=[jax.ShapeDtypeStruct((out_dim, e), jnp.float32)] * 2,
    )(edge_attr, w2, b2)

    p0 = _sc_round(g0t, eat0, src, dst)

    hv1t, g1t = pl.pallas_call(
        _mid_body,
        out_shape=[jax.ShapeDtypeStruct((out_dim, n), jnp.float32)] * 2,
    )(p0, hv0t, W_u1[:out_dim])

    p1 = _sc_round(g1t, eat1, src, dst)

    out = pl.pallas_call(
        _fin_body,
        out_shape=jax.ShapeDtypeStruct((n, out_dim), jnp.float32),
    )(p1, hv1t, W_r, b_r.reshape(1, -1))

    return out


# speedup vs baseline: 1.7459x; 1.0230x over previous
import functools

import jax
import jax.numpy as jnp
from jax import lax
from jax.experimental import pallas as pl
from jax.experimental.pallas import tpu as pltpu
from jax.experimental.pallas import tpu_sc as plsc

_DN0 = (((0,), (0,)), ((), ()))
_DN01 = (((0,), (1,)), ((), ()))

_EDGE_BLOCK = 16000
_SC_WINDOW = 3200


def _prep_nodes_body(x_ref, w_in_ref, b_in_ref, w_top_ref, hv0t_ref, g0t_ref):
    hv0t = lax.dot_general(
        w_in_ref[...], x_ref[...], _DN01, preferred_element_type=jnp.float32
    ) + b_in_ref[...]
    hv0t_ref[...] = hv0t
    g0t_ref[...] = lax.dot_general(
        w_top_ref[...], hv0t, _DN0, preferred_element_type=jnp.float32
    )


def _prep_edges_body(ea_ref, w2_ref, b2_ref, eat0_ref, eat1_ref):
    m = lax.dot_general(
        w2_ref[...], ea_ref[...], _DN01, preferred_element_type=jnp.float32
    ) + b2_ref[...]
    half = m.shape[0] // 2
    eat0_ref[...] = m[:half]
    eat1_ref[...] = m[half:]


def _mid_body(pt_ref, hvt_ref, w_top_ref, hv1t_ref, g1t_ref):
    pooled = jnp.max(pt_ref[...], axis=0)[:, : hvt_ref.shape[1]]
    hv1t = jnp.maximum(pooled, 0.0) + hvt_ref[...]
    hv1t_ref[...] = hv1t
    g1t_ref[...] = lax.dot_general(
        w_top_ref[...], hv1t, _DN0, preferred_element_type=jnp.float32
    )


def _fin_body(pt_ref, hvt_ref, w_r_ref, b_r_ref, out_ref):
    pooled = jnp.max(pt_ref[...], axis=0)[:, : hvt_ref.shape[1]]
    hv2t = jnp.maximum(pooled, 0.0) + hvt_ref[...]
    out_ref[...] = jnp.maximum(
        lax.dot_general(hv2t, w_r_ref[...], _DN0, preferred_element_type=jnp.float32)
        + b_r_ref[...],
        0.0,
    )


_K = 4
_C = 4

_GDN = lax.GatherDimensionNumbers(
    offset_dims=(), collapsed_slice_dims=(0,), start_index_map=(0,)
)


def _take(x, idx):
    return lax.gather(
        x,
        idx[:, None],
        dimension_numbers=_GDN,
        slice_sizes=(1,),
        mode=lax.GatherScatterMode.PROMISE_IN_BOUNDS,
    )


def _sc_round(gt, eat, src, dst):
    n = gt.shape[1]
    e = src.shape[0]
    w = _SC_WINDOW
    ec = e // _C
    mesh = plsc.VectorSubcoreMesh(core_axis_name="c", subcore_axis_name="s")

    @functools.partial(
        pl.kernel,
        out_type=jax.ShapeDtypeStruct((_C * 32, n + 16), jnp.float32),
        mesh=mesh,
        compiler_params=pltpu.CompilerParams(needs_layout_passes=False),
        scratch_types=[pltpu.VMEM((n,), jnp.float32)] * 4
        + [pltpu.VMEM((n + 16,), jnp.float32)] * 4
        + [pltpu.VMEM((w,), jnp.int32)] * 3
        + [pltpu.VMEM((w,), jnp.float32)] * 8
        + [pltpu.SemaphoreType.DMA],
    )
    def k(gt_h, eat_h, src_h, dst_h, out_h, *scr):
        gs = scr[0:4]
        accs = scr[4:8]
        src_v, dst_v, sdst_v = scr[8:11]
        eas = scr[11:15]
        zs = scr[15:19]
        sem = scr[19]
        c = lax.axis_index("c")
        s = lax.axis_index("s")
        wid = s * 2 + c
        grp = wid // _C
        j = wid % _C
        f0 = grp * _K

        for ki in range(_K):
            pltpu.sync_copy(gt_h.at[f0 + ki], gs[ki])

        neg_inf = jnp.full((16,), -jnp.inf, dtype=jnp.float32)

        @plsc.parallel_loop(0, (n + 16) // 16, unroll=4)
        def _init(i):
            for ki in range(_K):
                accs[ki][pl.ds(i * 16, 16)] = neg_inf

        lane = lax.iota(jnp.int32, 16)
        _SHIFT_UP = jnp.minimum(lane + 1, 15)
        _SHIFT_DN = {sh: jnp.maximum(lane - sh, 0) for sh in (1, 2, 4, 8)}
        base0 = j * ec

        def win_body(wi, carry):
            base = pl.multiple_of(base0 + wi * w, 128)
            cps = [
                pltpu.async_copy(src_h.at[pl.ds(base, w)], src_v, sem),
                pltpu.async_copy(dst_h.at[pl.ds(base, w)], dst_v, sem),
            ]
            for ki in range(_K):
                cps.append(
                    pltpu.async_copy(eat_h.at[f0 + ki, pl.ds(base, w)], eas[ki], sem)
                )
            for cp in cps:
                cp.wait()

            @plsc.parallel_loop(0, w // 16, unroll=4)
            def _zloop(i):
                off = i * 16
                sv = src_v[pl.ds(off, 16)]
                dv = dst_v[pl.ds(off, 16)]
                sd, p = plsc.sort_key_val(dv, lane)
                nxt = _take(sd, _SHIFT_UP)
                is_end = (sd != nxt) | (lane == 15)
                sdst_v[pl.ds(off, 16)] = jnp.where(is_end, sd, n + lane)
                eqs = []
                for sh in (1, 2, 4, 8):
                    prev = _take(sd, _SHIFT_DN[sh])
                    eqs.append((sd == prev) & (lane >= sh))
                for ki in range(_K):
                    z = plsc.load_gather(gs[ki], [sv]) + eas[ki][pl.ds(off, 16)]
                    zp = _take(z, p)
                    for sh, eq in zip((1, 2, 4, 8), eqs):
                        zsh = _take(zp, _SHIFT_DN[sh])
                        zp = jnp.maximum(zp, jnp.where(eq, zsh, -jnp.inf))
                    zs[ki][pl.ds(off, 16)] = zp

            def vec_body(i, carry2):
                for u in range(2):
                    off = (i * 2 + u) * 16
                    sd = sdst_v[pl.ds(off, 16)]
                    for ki in range(_K):
                        zf = zs[ki][pl.ds(off, 16)]
                        cur = plsc.load_gather(accs[ki], [sd])
                        plsc.store_scatter(accs[ki], [sd], jnp.maximum(cur, zf))
                return carry2

            lax.fori_loop(0, w // 32, vec_body, 0)
            return carry

        lax.fori_loop(0, ec // w, win_body, 0)
        for ki in range(_K):
            pltpu.sync_copy(accs[ki], out_h.at[j * 32 + f0 + ki])

    return k(gt, eat, src, dst).reshape(_C, 32, n + 16)


def kernel(x, edge_index, edge_attr, W_in, b_in, W_u0, b_u0, W_u1, b_u1, W_r, b_r):
    n, _ = x.shape
    e = edge_index.shape[1]
    out_dim = W_in.shape[1]

    src = edge_index[0].astype(jnp.int32)
    dst = edge_index[1].astype(jnp.int32)

    w2 = jnp.concatenate([W_u0[out_dim:], W_u1[out_dim:]], axis=1)
    b2 = jnp.concatenate([b_u0, b_u1]).reshape(-1, 1)

    hv0t, g0t = pl.pallas_call(
        _prep_nodes_body,
        out_shape=[jax.ShapeDtypeStruct((out_dim, n), jnp.float32)] * 2,
    )(x, W_in, b_in.reshape(-1, 1), W_u0[:out_dim])

    de = edge_attr.shape[1]
    blk = _EDGE_BLOCK
    eat0, eat1 = pl.pallas_call(
        _prep_edges_body,
        grid=(e // blk,),
        in_specs=[
            pl.BlockSpec((blk, de), lambda i: (i, 0)),
            pl.BlockSpec((de, 2 * out_dim), lambda i: (0, 0)),
            pl.BlockSpec((2 * out_dim, 1), lambda i: (0, 0)),
        ],
        out_specs=[
            pl.BlockSpec((out_dim, blk), lambda i: (0, i)),
            pl.BlockSpec((out_dim, blk), lambda i: (0, i)),
        ],
        out_shape=[jax.ShapeDtypeStruct((out_dim, e), jnp.float32)] * 2,
    )(edge_attr, w2, b2)

    p0 = _sc_round(g0t, eat0, src, dst)

    hv1t, g1t = pl.pallas_call(
        _mid_body,
        out_shape=[jax.ShapeDtypeStruct((out_dim, n), jnp.float32)] * 2,
    )(p0, hv0t, W_u1[:out_dim])

    p1 = _sc_round(g1t, eat1, src, dst)

    out = pl.pallas_call(
        _fin_body,
        out_shape=jax.ShapeDtypeStruct((n, out_dim), jnp.float32),
    )(p1, hv1t, W_r, b_r.reshape(1, -1))

    return out
